# baseline (speedup 1.0000x reference)
import jax
import jax.numpy as jnp
from jax.experimental import pallas as pl
from jax.experimental.pallas import tpu as pltpu

EMOJI_VOCAB = 100
EMOJI_DIM = 50
SPACY_DIM = 300
OUT_DIM = SPACY_DIM + EMOJI_DIM

_TILE = 128
_HEAD = 256
_TAIL_OFF = SPACY_DIM - _HEAD
_BB = 64
_NCOPY = 8


def _tail_kernel(spacy_ref, ids_ref, table_ref, out_ref):
    bb, seq = ids_ref.shape
    out_ref[:, :, :_TAIL_OFF] = spacy_ref[:, :, :_TAIL_OFF]
    table = table_ref[...]
    for i in range(bb):
        vocab_iota = jax.lax.broadcasted_iota(
            jnp.int32, (seq, EMOJI_VOCAB), 1)
        onehot = (ids_ref[i, :][:, None] == vocab_iota).astype(jnp.float32)
        emoji = jax.lax.dot_general(
            onehot, table, (((1,), (0,)), ((), ())),
            preferred_element_type=jnp.float32)
        out_ref[i, :, _TAIL_OFF:_TAIL_OFF + EMOJI_DIM] = emoji


def _head_copy_kernel(spacy_ref, outa_ref, out_ref, sem):
    del outa_ref
    b = spacy_ref.shape[0]
    chunk = b // _NCOPY
    copies = [
        pltpu.make_async_copy(
            spacy_ref.at[pl.ds(k * chunk, chunk), :, pl.ds(0, _HEAD)],
            out_ref.at[pl.ds(k * chunk, chunk), :, pl.ds(0, _HEAD)],
            sem)
        for k in range(_NCOPY)
    ]
    for cp in copies:
        cp.start()
    for cp in copies:
        cp.wait()


def kernel(spacy_vectors, emoji_ids, emoji_table):
    b, s, d = spacy_vectors.shape
    out_shape = jax.ShapeDtypeStruct((b, s, OUT_DIM), jnp.float32)

    tail = pl.pallas_call(
        _tail_kernel,
        grid=(b // _BB,),
        in_specs=[
            pl.BlockSpec((_BB, s, _TILE), lambda i: (i, 0, _HEAD // _TILE)),
            pl.BlockSpec((_BB, s), lambda i: (i, 0)),
            pl.BlockSpec((EMOJI_VOCAB, EMOJI_DIM), lambda i: (0, 0)),
        ],
        out_specs=pl.BlockSpec((_BB, s, _TILE), lambda i: (i, 0, _HEAD // _TILE)),
        out_shape=out_shape,
    )(spacy_vectors, emoji_ids, emoji_table)

    return pl.pallas_call(
        _head_copy_kernel,
        in_specs=[
            pl.BlockSpec(memory_space=pl.ANY),
            pl.BlockSpec(memory_space=pl.ANY),
        ],
        out_specs=pl.BlockSpec(memory_space=pl.ANY),
        out_shape=out_shape,
        scratch_shapes=[pltpu.SemaphoreType.DMA],
        input_output_aliases={1: 0},
    )(spacy_vectors, tail)


# speedup vs baseline: 7.6610x; 7.6610x over previous
import functools

import jax
import jax.numpy as jnp
from jax import lax
from jax.experimental import pallas as pl
from jax.experimental.pallas import tpu as pltpu
from jax.experimental.pallas import tpu_sc as plsc

EMOJI_VOCAB = 100
EMOJI_DIM = 50
SPACY_DIM = 300
OUT_DIM = SPACY_DIM + EMOJI_DIM

_TILE = 128
_HEAD = 256
_TAIL_OFF = SPACY_DIM - _HEAD
_BB = 64
_NUM_WORKERS = 32
_CB = 4


def _make_head_copy(b, s):
    per_w = b // _NUM_WORKERS
    n_chunks = per_w // _CB
    mesh = plsc.VectorSubcoreMesh(core_axis_name="c", subcore_axis_name="s")

    @functools.partial(
        pl.kernel,
        mesh=mesh,
        out_type=jax.ShapeDtypeStruct((b, s, OUT_DIM), jnp.float32),
        scratch_types=[
            pltpu.VMEM((_CB, s, _HEAD), jnp.float32),
        ],
    )
    def head_copy(spacy_hbm, out_hbm, buf):
        wid = lax.axis_index("s") * 2 + lax.axis_index("c")
        w_base = wid * per_w

        def body(k, carry):
            b0 = w_base + k * _CB
            pltpu.sync_copy(
                spacy_hbm.at[pl.ds(b0, _CB), :, pl.ds(0, _HEAD)], buf)
            pltpu.sync_copy(
                buf, out_hbm.at[pl.ds(b0, _CB), :, pl.ds(0, _HEAD)])
            return carry

        lax.fori_loop(0, n_chunks, body, 0)

    return head_copy


def _tail_kernel(spacy_ref, ids_ref, table_ref, outa_ref, out_ref):
    del outa_ref
    bb, seq = ids_ref.shape
    out_ref[:, :, :_TAIL_OFF] = spacy_ref[:, :, :_TAIL_OFF]
    table = table_ref[...]
    for i in range(bb):
        vocab_iota = jax.lax.broadcasted_iota(
            jnp.int32, (seq, EMOJI_VOCAB), 1)
        onehot = (ids_ref[i, :][:, None] == vocab_iota).astype(jnp.float32)
        emoji = jax.lax.dot_general(
            onehot, table, (((1,), (0,)), ((), ())),
            preferred_element_type=jnp.float32)
        out_ref[i, :, _TAIL_OFF:_TAIL_OFF + EMOJI_DIM] = emoji


def kernel(spacy_vectors, emoji_ids, emoji_table):
    b, s, d = spacy_vectors.shape
    out_shape = jax.ShapeDtypeStruct((b, s, OUT_DIM), jnp.float32)

    head = _make_head_copy(b, s)(spacy_vectors)

    return pl.pallas_call(
        _tail_kernel,
        grid=(b // _BB,),
        in_specs=[
            pl.BlockSpec((_BB, s, _TILE), lambda i: (i, 0, _HEAD // _TILE)),
            pl.BlockSpec((_BB, s), lambda i: (i, 0)),
            pl.BlockSpec((EMOJI_VOCAB, EMOJI_DIM), lambda i: (0, 0)),
            pl.BlockSpec(memory_space=pl.ANY),
        ],
        out_specs=pl.BlockSpec((_BB, s, _TILE), lambda i: (i, 0, _HEAD // _TILE)),
        out_shape=out_shape,
        input_output_aliases={3: 0},
    )(spacy_vectors, emoji_ids, emoji_table, head)
